# Initial kernel scaffold; baseline (speedup 1.0000x reference)
import functools

import jax
import jax.numpy as jnp
from jax import lax
from jax.experimental import pallas as pl
from jax.experimental.pallas import tpu as pltpu
from jax.experimental.pallas import tpu_sc as plsc

N = 100000
E = 1600000
D = 128
EDIM = 16
K = 20
B = 4096
H = 2
DH = D // H
B3 = 3 * B
NC = 2
NS = 16
NW = NC * NS
LANE = 128
RDIM = B3 // LANE
RPW = RDIM // NW

QBLK = 128


def _add_body(a_ref, b_ref, o_ref):
    o_ref[...] = a_ref[...] + b_ref[...]


def _fused_table(node_features, memory):
    rows = node_features.shape[0]
    blk = 2048
    grid = (rows + blk - 1) // blk
    return pl.pallas_call(
        _add_body,
        grid=(grid,),
        in_specs=[
            pl.BlockSpec((blk, D), lambda i: (i, 0)),
            pl.BlockSpec((blk, D), lambda i: (i, 0)),
        ],
        out_specs=pl.BlockSpec((blk, D), lambda i: (i, 0)),
        out_shape=jax.ShapeDtypeStruct((rows, D), jnp.float32),
    )(node_features, memory)


def _sc_gather_body(fused_hbm, etab_hbm, ngh3_hbm, eidx3_hbm, nodes3_hbm,
                    ngh_out, ef_out, src_out,
                    idx_v, rows_v, efrows_v, sem):
    wid = lax.axis_index("s") * NC + lax.axis_index("c")
    base = wid * RPW

    pltpu.sync_copy(nodes3_hbm.at[pl.ds(base, RPW)], idx_v)
    for c in range(RPW):
        pltpu.async_copy(fused_hbm.at[idx_v.at[c]], rows_v.at[c], sem).wait()
    pltpu.sync_copy(rows_v, src_out.at[pl.ds(base, RPW)])

    def body(n, _):
        pltpu.sync_copy(ngh3_hbm.at[n, pl.ds(base, RPW)], idx_v)
        for c in range(RPW):
            pltpu.async_copy(fused_hbm.at[idx_v.at[c]], rows_v.at[c], sem).wait()
        pltpu.sync_copy(rows_v, ngh_out.at[n, pl.ds(base, RPW)])

        pltpu.sync_copy(eidx3_hbm.at[n, pl.ds(base, RPW)], idx_v)
        for c in range(RPW):
            pltpu.async_copy(etab_hbm.at[idx_v.at[c]], efrows_v.at[c], sem).wait()
        pltpu.sync_copy(efrows_v, ef_out.at[n, pl.ds(base, RPW)])
        return 0

    lax.fori_loop(0, K, body, 0)


def _sc_gather(fused, edge_features, ngh3, eidx3, nodes3):
    mesh = plsc.VectorSubcoreMesh(core_axis_name="c", subcore_axis_name="s")
    f = pl.kernel(
        _sc_gather_body,
        out_type=[
            jax.ShapeDtypeStruct((K, RDIM, LANE, D), jnp.float32),
            jax.ShapeDtypeStruct((K, RDIM, LANE, EDIM), jnp.float32),
            jax.ShapeDtypeStruct((RDIM, LANE, D), jnp.float32),
        ],
        mesh=mesh,
        scratch_types=[
            pltpu.VMEM((RPW, LANE), jnp.int32),
            pltpu.VMEM((RPW, LANE, D), jnp.float32),
            pltpu.VMEM((RPW, LANE, EDIM), jnp.float32),
            pltpu.SemaphoreType.DMA,
        ],
    )
    return f(fused, edge_features, ngh3, eidx3, nodes3)


def _dense_body(ngh_ref, ef_ref, src_ref, dt_ref, nodes_ref,
                tw_ref, tb_ref, wq_ref, wk_ref, wv_ref, w1_ref, b1_ref,
                w2_ref, b2_ref, out_ref):
    q_n = QBLK
    tw = tw_ref[...]
    tb = tb_ref[...]

    dt = dt_ref[...]
    te = jnp.cos(dt[:, :, None] * tw[0][None, None, :] + tb[0][None, None, :])

    ngh2 = ngh_ref[...].reshape(K * q_n, D)
    ef2 = ef_ref[...].reshape(K * q_n, EDIM)
    te2 = te.reshape(K * q_n, D)

    wk0 = wk_ref[0:D, :]
    wk1 = wk_ref[D:D + EDIM, :]
    wk2 = wk_ref[D + EDIM:D + EDIM + D, :]
    wv0 = wv_ref[0:D, :]
    wv1 = wv_ref[D:D + EDIM, :]
    wv2 = wv_ref[D + EDIM:D + EDIM + D, :]

    dot = functools.partial(jnp.dot, preferred_element_type=jnp.float32)
    kmat = dot(ngh2, wk0) + dot(ef2, wk1) + dot(te2, wk2)
    vmat = dot(ngh2, wv0) + dot(ef2, wv1) + dot(te2, wv2)

    src = src_ref[...]
    qbias = dot(jnp.cos(tb), wq_ref[D:2 * D, :])
    q = dot(src, wq_ref[0:D, :]) + qbias

    k3 = kmat.reshape(K, q_n, D)
    v3 = vmat.reshape(K, q_n, D)

    lane = lax.broadcasted_iota(jnp.int32, (1, 1, D), 2)
    m0 = (lane < DH).astype(jnp.float32)

    prod = q[None, :, :] * k3
    scale = 1.0 / jnp.sqrt(jnp.float32(DH))
    l_all = jnp.sum(prod, axis=-1) * scale
    l0 = jnp.sum(prod * m0, axis=-1) * scale
    l1 = l_all - l0

    invalid = nodes_ref[...] == 0
    neg = jnp.float32(-1e10)
    l0 = jnp.where(invalid, neg, l0)
    l1 = jnp.where(invalid, neg, l1)

    def softmax0(l):
        m = jnp.max(l, axis=0, keepdims=True)
        e = jnp.exp(l - m)
        return e / jnp.sum(e, axis=0, keepdims=True)

    a0 = softmax0(l0)
    a1 = softmax0(l1)
    af = a0[:, :, None] * m0 + a1[:, :, None] * (1.0 - m0)
    attn_out = jnp.sum(af * v3, axis=0)

    hmid = jnp.maximum(
        dot(attn_out, w1_ref[0:D, :]) + dot(src, w1_ref[D:2 * D, :]) + b1_ref[...],
        0.0)
    out_ref[...] = dot(hmid, w2_ref[...]) + b2_ref[...]


def _dense_stage(ngh_feat, ef, src_feat, dtT, nodesT,
                 time_w, time_b, Wq, Wk, Wv, W1, b1, W2, b2):
    grid = (B3 // QBLK,)

    def wspec(shape):
        nd = len(shape)
        return pl.BlockSpec(shape, lambda i, _nd=nd: (0,) * _nd)

    return pl.pallas_call(
        _dense_body,
        grid=grid,
        in_specs=[
            pl.BlockSpec((K, QBLK, D), lambda i: (0, i, 0)),
            pl.BlockSpec((K, QBLK, EDIM), lambda i: (0, i, 0)),
            pl.BlockSpec((QBLK, D), lambda i: (i, 0)),
            pl.BlockSpec((K, QBLK), lambda i: (0, i)),
            pl.BlockSpec((K, QBLK), lambda i: (0, i)),
            wspec((1, D)),
            wspec((1, D)),
            wspec((2 * D, D)),
            wspec((2 * D + EDIM, D)),
            wspec((2 * D + EDIM, D)),
            wspec((2 * D, D)),
            wspec((1, D)),
            wspec((D, D)),
            wspec((1, D)),
        ],
        out_specs=pl.BlockSpec((QBLK, D), lambda i: (i, 0)),
        out_shape=jax.ShapeDtypeStruct((B3, D), jnp.float32),
    )(ngh_feat, ef, src_feat, dtT, nodesT,
      time_w.reshape(1, D), time_b.reshape(1, D), Wq, Wk, Wv,
      W1, b1.reshape(1, D), W2, b2.reshape(1, D))


def kernel(node_features, edge_features, memory, src_idx, tgt_idx, bgd_idx,
           cut_time, e_idx, ngh_nodes, ngh_eidx, ngh_times, time_w, time_b,
           Wq, Wk, Wv, W1, b1, W2, b2):
    fused = _fused_table(node_features, memory)

    nodes = jnp.concatenate([src_idx, tgt_idx, bgd_idx], axis=0)
    nodes3 = nodes.reshape(RDIM, LANE)
    nghT = ngh_nodes.T
    ngh3 = nghT.reshape(K, RDIM, LANE)
    eidx3 = ngh_eidx.T.reshape(K, RDIM, LANE)

    ngh_feat4, ef4, src3 = _sc_gather(fused, edge_features, ngh3, eidx3, nodes3)
    ngh_feat = ngh_feat4.reshape(K, B3, D)
    ef = ef4.reshape(K, B3, EDIM)
    src_feat = src3.reshape(B3, D)

    ts = jnp.concatenate([cut_time, cut_time, cut_time], axis=0)
    dtT = ts[None, :] - ngh_times.T

    return _dense_stage(ngh_feat, ef, src_feat, dtT, nghT,
                        time_w, time_b, Wq, Wk, Wv, W1, b1, W2, b2)


# speedup vs baseline: 4.6222x; 4.6222x over previous
import functools

import jax
import jax.numpy as jnp
from jax import lax
from jax.experimental import pallas as pl
from jax.experimental.pallas import tpu as pltpu
from jax.experimental.pallas import tpu_sc as plsc

N = 100000
E = 1600000
D = 128
EDIM = 16
K = 20
B = 4096
H = 2
DH = D // H
B3 = 3 * B
NC = 2
NS = 16
NW = NC * NS
LANE = 128
RDIM = B3 // LANE
RPW = RDIM // NW

QBLK = 128


def _add_body(a_ref, b_ref, o_ref):
    o_ref[...] = a_ref[...] + b_ref[...]


def _fused_table(node_features, memory):
    rows = node_features.shape[0]
    blk = 2048
    grid = (rows + blk - 1) // blk
    return pl.pallas_call(
        _add_body,
        grid=(grid,),
        in_specs=[
            pl.BlockSpec((blk, D), lambda i: (i, 0)),
            pl.BlockSpec((blk, D), lambda i: (i, 0)),
        ],
        out_specs=pl.BlockSpec((blk, D), lambda i: (i, 0)),
        out_shape=jax.ShapeDtypeStruct((rows, D), jnp.float32),
    )(node_features, memory)


def _sc_gather_body(fused_hbm, etab_hbm, ngh3_hbm, eidx3_hbm, nodes3_hbm,
                    ngh_out, ef_out, src_out,
                    idx_v, rows_v, efrows_v, sem):
    wid = lax.axis_index("s") * NC + lax.axis_index("c")
    base = wid * RPW

    pltpu.sync_copy(nodes3_hbm.at[wid], idx_v)
    for c in range(RPW):
        pltpu.async_copy(fused_hbm.at[idx_v.at[c]], rows_v.at[c], sem).wait()
    pltpu.sync_copy(rows_v, src_out.at[pl.ds(base, RPW)])

    def body(n, _):
        pltpu.sync_copy(ngh3_hbm.at[n, wid], idx_v)
        for c in range(RPW):
            pltpu.async_copy(fused_hbm.at[idx_v.at[c]], rows_v.at[c], sem).wait()
        pltpu.sync_copy(rows_v, ngh_out.at[n, pl.ds(base, RPW)])

        pltpu.sync_copy(eidx3_hbm.at[n, wid], idx_v)
        for c in range(RPW):
            pltpu.async_copy(etab_hbm.at[idx_v.at[c]], efrows_v.at[c], sem).wait()
        pltpu.sync_copy(efrows_v, ef_out.at[n, pl.ds(base, RPW)])
        return 0

    lax.fori_loop(0, K, body, 0)


def _sc_gather(fused, edge_features, ngh3, eidx3, nodes3):
    mesh = plsc.VectorSubcoreMesh(core_axis_name="c", subcore_axis_name="s")
    f = pl.kernel(
        _sc_gather_body,
        out_type=[
            jax.ShapeDtypeStruct((K, RDIM, LANE, D), jnp.float32),
            jax.ShapeDtypeStruct((K, RDIM, LANE, EDIM), jnp.float32),
            jax.ShapeDtypeStruct((RDIM, LANE, D), jnp.float32),
        ],
        mesh=mesh,
        scratch_types=[
            pltpu.VMEM((RPW, LANE), jnp.int32),
            pltpu.VMEM((RPW, LANE, D), jnp.float32),
            pltpu.VMEM((RPW, LANE, EDIM), jnp.float32),
            pltpu.SemaphoreType.DMA,
        ],
        compiler_params=pltpu.CompilerParams(use_tc_tiling_on_sc=False),
    )
    return f(fused, edge_features, ngh3, eidx3, nodes3)


def _dense_body(ngh_ref, ef_ref, src_ref, dt_ref, nodes_ref,
                tw_ref, tb_ref, wq_ref, wk_ref, wv_ref, w1_ref, b1_ref,
                w2_ref, b2_ref, out_ref):
    q_n = QBLK
    tw = tw_ref[...]
    tb = tb_ref[...]

    dt = dt_ref[...]
    te = jnp.cos(dt[:, :, None] * tw[0][None, None, :] + tb[0][None, None, :])

    ngh2 = ngh_ref[...].reshape(K * q_n, D)
    ef2 = ef_ref[...].reshape(K * q_n, EDIM)
    te2 = te.reshape(K * q_n, D)

    wk0 = wk_ref[0:D, :]
    wk1 = wk_ref[D:D + EDIM, :]
    wk2 = wk_ref[D + EDIM:D + EDIM + D, :]
    wv0 = wv_ref[0:D, :]
    wv1 = wv_ref[D:D + EDIM, :]
    wv2 = wv_ref[D + EDIM:D + EDIM + D, :]

    dot = functools.partial(jnp.dot, preferred_element_type=jnp.float32)
    kmat = dot(ngh2, wk0) + dot(ef2, wk1) + dot(te2, wk2)
    vmat = dot(ngh2, wv0) + dot(ef2, wv1) + dot(te2, wv2)

    src = src_ref[...]
    qbias = dot(jnp.cos(tb), wq_ref[D:2 * D, :])
    q = dot(src, wq_ref[0:D, :]) + qbias

    k3 = kmat.reshape(K, q_n, D)
    v3 = vmat.reshape(K, q_n, D)

    lane = lax.broadcasted_iota(jnp.int32, (1, 1, D), 2)
    m0 = (lane < DH).astype(jnp.float32)

    prod = q[None, :, :] * k3
    scale = 1.0 / jnp.sqrt(jnp.float32(DH))
    l_all = jnp.sum(prod, axis=-1) * scale
    l0 = jnp.sum(prod * m0, axis=-1) * scale
    l1 = l_all - l0

    invalid = nodes_ref[...] == 0
    neg = jnp.float32(-1e10)
    l0 = jnp.where(invalid, neg, l0)
    l1 = jnp.where(invalid, neg, l1)

    def softmax0(l):
        m = jnp.max(l, axis=0, keepdims=True)
        e = jnp.exp(l - m)
        return e / jnp.sum(e, axis=0, keepdims=True)

    a0 = softmax0(l0)
    a1 = softmax0(l1)
    af = a0[:, :, None] * m0 + a1[:, :, None] * (1.0 - m0)
    attn_out = jnp.sum(af * v3, axis=0)

    hmid = jnp.maximum(
        dot(attn_out, w1_ref[0:D, :]) + dot(src, w1_ref[D:2 * D, :]) + b1_ref[...],
        0.0)
    out_ref[...] = dot(hmid, w2_ref[...]) + b2_ref[...]


def _dense_stage(ngh_feat, ef, src_feat, dtT, nodesT,
                 time_w, time_b, Wq, Wk, Wv, W1, b1, W2, b2):
    grid = (B3 // QBLK,)

    def wspec(shape):
        nd = len(shape)
        return pl.BlockSpec(shape, lambda i, _nd=nd: (0,) * _nd)

    return pl.pallas_call(
        _dense_body,
        grid=grid,
        in_specs=[
            pl.BlockSpec((K, QBLK, D), lambda i: (0, i, 0)),
            pl.BlockSpec((K, QBLK, EDIM), lambda i: (0, i, 0)),
            pl.BlockSpec((QBLK, D), lambda i: (i, 0)),
            pl.BlockSpec((K, QBLK), lambda i: (0, i)),
            pl.BlockSpec((K, QBLK), lambda i: (0, i)),
            wspec((1, D)),
            wspec((1, D)),
            wspec((2 * D, D)),
            wspec((2 * D + EDIM, D)),
            wspec((2 * D + EDIM, D)),
            wspec((2 * D, D)),
            wspec((1, D)),
            wspec((D, D)),
            wspec((1, D)),
        ],
        out_specs=pl.BlockSpec((QBLK, D), lambda i: (i, 0)),
        out_shape=jax.ShapeDtypeStruct((B3, D), jnp.float32),
    )(ngh_feat, ef, src_feat, dtT, nodesT,
      time_w.reshape(1, D), time_b.reshape(1, D), Wq, Wk, Wv,
      W1, b1.reshape(1, D), W2, b2.reshape(1, D))


def kernel(node_features, edge_features, memory, src_idx, tgt_idx, bgd_idx,
           cut_time, e_idx, ngh_nodes, ngh_eidx, ngh_times, time_w, time_b,
           Wq, Wk, Wv, W1, b1, W2, b2):
    fused = _fused_table(node_features, memory)

    nodes = jnp.concatenate([src_idx, tgt_idx, bgd_idx], axis=0)
    nodes3 = nodes.reshape(NW, RPW, LANE)
    nghT = ngh_nodes.T
    ngh3 = nghT.reshape(K, NW, RPW, LANE)
    eidx3 = ngh_eidx.T.reshape(K, NW, RPW, LANE)

    ngh_feat4, ef4, src3 = _sc_gather(fused, edge_features, ngh3, eidx3, nodes3)
    ngh_feat = ngh_feat4.reshape(K, B3, D)
    ef = ef4.reshape(K, B3, EDIM)
    src_feat = src3.reshape(B3, D)

    ts = jnp.concatenate([cut_time, cut_time, cut_time], axis=0)
    dtT = ts[None, :] - ngh_times.T

    return _dense_stage(ngh_feat, ef, src_feat, dtT, nghT,
                        time_w, time_b, Wq, Wk, Wv, W1, b1, W2, b2)


# speedup vs baseline: 5.7399x; 1.2418x over previous
import functools

import jax
import jax.numpy as jnp
from jax import lax
from jax.experimental import pallas as pl
from jax.experimental.pallas import tpu as pltpu
from jax.experimental.pallas import tpu_sc as plsc

N = 100000
E = 1600000
D = 128
EDIM = 16
K = 20
B = 4096
H = 2
DH = D // H
B3 = 3 * B
NC = 2
NS = 16
NW = NC * NS
LANE = 128
RDIM = B3 // LANE
RPW = RDIM // NW

QBLK = 128


def _add_body(a_ref, b_ref, o_ref):
    o_ref[...] = a_ref[...] + b_ref[...]


def _fused_table(node_features, memory):
    rows = node_features.shape[0]
    blk = 2048
    grid = (rows + blk - 1) // blk
    return pl.pallas_call(
        _add_body,
        grid=(grid,),
        in_specs=[
            pl.BlockSpec((blk, D), lambda i: (i, 0)),
            pl.BlockSpec((blk, D), lambda i: (i, 0)),
        ],
        out_specs=pl.BlockSpec((blk, D), lambda i: (i, 0)),
        out_shape=jax.ShapeDtypeStruct((rows, D), jnp.float32),
    )(node_features, memory)


def _sc_gather_body(fused_hbm, etab_hbm, ngh3_hbm, eidx3_hbm, nodes3_hbm,
                    ngh_out, ef_out, src_out,
                    idx_v, rows_v, efrows_v, sem):
    wid = lax.axis_index("s") * NC + lax.axis_index("c")
    base = wid * RPW * LANE

    pltpu.sync_copy(nodes3_hbm.at[wid], idx_v)
    for c in range(RPW):
        pltpu.async_copy(fused_hbm.at[idx_v.at[c]],
                         rows_v.at[pl.ds(c * LANE, LANE)], sem).wait()
    pltpu.sync_copy(rows_v, src_out.at[pl.ds(base, RPW * LANE)])

    def body(n, _):
        pltpu.sync_copy(ngh3_hbm.at[n, wid], idx_v)
        for c in range(RPW):
            pltpu.async_copy(fused_hbm.at[idx_v.at[c]],
                             rows_v.at[pl.ds(c * LANE, LANE)], sem).wait()
        pltpu.sync_copy(rows_v, ngh_out.at[n, pl.ds(base, RPW * LANE)])

        pltpu.sync_copy(eidx3_hbm.at[n, wid], idx_v)
        for c in range(RPW):
            pltpu.async_copy(etab_hbm.at[idx_v.at[c]],
                             efrows_v.at[pl.ds(c * LANE, LANE)], sem).wait()
        pltpu.sync_copy(efrows_v, ef_out.at[n, pl.ds(base, RPW * LANE)])
        return 0

    lax.fori_loop(0, K, body, 0)


def _sc_gather(fused, edge_features, ngh3, eidx3, nodes3):
    mesh = plsc.VectorSubcoreMesh(core_axis_name="c", subcore_axis_name="s")
    f = pl.kernel(
        _sc_gather_body,
        out_type=[
            jax.ShapeDtypeStruct((K, B3, D), jnp.float32),
            jax.ShapeDtypeStruct((K, B3, EDIM), jnp.float32),
            jax.ShapeDtypeStruct((B3, D), jnp.float32),
        ],
        mesh=mesh,
        scratch_types=[
            pltpu.VMEM((RPW, LANE), jnp.int32),
            pltpu.VMEM((RPW * LANE, D), jnp.float32),
            pltpu.VMEM((RPW * LANE, EDIM), jnp.float32),
            pltpu.SemaphoreType.DMA,
        ],
        compiler_params=pltpu.CompilerParams(use_tc_tiling_on_sc=False),
    )
    return f(fused, edge_features, ngh3, eidx3, nodes3)


_INV2PI = 0.15915494309189535
_COS_C = (0.99999998, -19.73920414, 64.93912897, -85.45061558,
          60.17266892, -25.9887977, 6.56003795)


def _fast_cos(x):
    r = x * _INV2PI
    r = r - jnp.round(r)
    u = r * r
    p = jnp.float32(_COS_C[6])
    for c in (_COS_C[5], _COS_C[4], _COS_C[3], _COS_C[2], _COS_C[1], _COS_C[0]):
        p = p * u + jnp.float32(c)
    return p


def _dense_body(ngh_ref, ef_ref, src_ref, dt_ref, nodes_ref,
                tw_ref, tb_ref, wq_ref, wk_ref, wv_ref, w1_ref, b1_ref,
                w2_ref, b2_ref, out_ref):
    q_n = QBLK
    tw = tw_ref[...]
    tb = tb_ref[...]

    dt = dt_ref[...]
    te = _fast_cos(dt[:, :, None] * tw[0][None, None, :] + tb[0][None, None, :])

    ngh2 = ngh_ref[...].reshape(K * q_n, D)
    ef2 = ef_ref[...].reshape(K * q_n, EDIM)
    te2 = te.reshape(K * q_n, D)

    wk0 = wk_ref[0:D, :]
    wk1 = wk_ref[D:D + EDIM, :]
    wk2 = wk_ref[D + EDIM:D + EDIM + D, :]
    wv0 = wv_ref[0:D, :]
    wv1 = wv_ref[D:D + EDIM, :]
    wv2 = wv_ref[D + EDIM:D + EDIM + D, :]

    dot = functools.partial(jnp.dot, preferred_element_type=jnp.float32)
    kmat = dot(ngh2, wk0) + dot(ef2, wk1) + dot(te2, wk2)
    vmat = dot(ngh2, wv0) + dot(ef2, wv1) + dot(te2, wv2)

    src = src_ref[...]
    qbias = dot(jnp.cos(tb), wq_ref[D:2 * D, :])
    q = dot(src, wq_ref[0:D, :]) + qbias

    k3 = kmat.reshape(K, q_n, D)
    v3 = vmat.reshape(K, q_n, D)

    lane = lax.broadcasted_iota(jnp.int32, (1, 1, D), 2)
    m0 = (lane < DH).astype(jnp.float32)

    prod = q[None, :, :] * k3
    scale = 1.0 / jnp.sqrt(jnp.float32(DH))
    l_all = jnp.sum(prod, axis=-1) * scale
    l0 = jnp.sum(prod * m0, axis=-1) * scale
    l1 = l_all - l0

    invalid = nodes_ref[...] == 0
    neg = jnp.float32(-1e10)
    l0 = jnp.where(invalid, neg, l0)
    l1 = jnp.where(invalid, neg, l1)

    def softmax0(l):
        m = jnp.max(l, axis=0, keepdims=True)
        e = jnp.exp(l - m)
        return e / jnp.sum(e, axis=0, keepdims=True)

    a0 = softmax0(l0)
    a1 = softmax0(l1)
    af = a0[:, :, None] * m0 + a1[:, :, None] * (1.0 - m0)
    attn_out = jnp.sum(af * v3, axis=0)

    hmid = jnp.maximum(
        dot(attn_out, w1_ref[0:D, :]) + dot(src, w1_ref[D:2 * D, :]) + b1_ref[...],
        0.0)
    out_ref[...] = dot(hmid, w2_ref[...]) + b2_ref[...]


def _dense_stage(ngh_feat, ef, src_feat, dtT, nodesT,
                 time_w, time_b, Wq, Wk, Wv, W1, b1, W2, b2):
    grid = (B3 // QBLK,)

    def wspec(shape):
        nd = len(shape)
        return pl.BlockSpec(shape, lambda i, _nd=nd: (0,) * _nd)

    return pl.pallas_call(
        _dense_body,
        grid=grid,
        in_specs=[
            pl.BlockSpec((K, QBLK, D), lambda i: (0, i, 0)),
            pl.BlockSpec((K, QBLK, EDIM), lambda i: (0, i, 0)),
            pl.BlockSpec((QBLK, D), lambda i: (i, 0)),
            pl.BlockSpec((K, QBLK), lambda i: (0, i)),
            pl.BlockSpec((K, QBLK), lambda i: (0, i)),
            wspec((1, D)),
            wspec((1, D)),
            wspec((2 * D, D)),
            wspec((2 * D + EDIM, D)),
            wspec((2 * D + EDIM, D)),
            wspec((2 * D, D)),
            wspec((1, D)),
            wspec((D, D)),
            wspec((1, D)),
        ],
        out_specs=pl.BlockSpec((QBLK, D), lambda i: (i, 0)),
        out_shape=jax.ShapeDtypeStruct((B3, D), jnp.float32),
    )(ngh_feat, ef, src_feat, dtT, nodesT,
      time_w.reshape(1, D), time_b.reshape(1, D), Wq, Wk, Wv,
      W1, b1.reshape(1, D), W2, b2.reshape(1, D))


def kernel(node_features, edge_features, memory, src_idx, tgt_idx, bgd_idx,
           cut_time, e_idx, ngh_nodes, ngh_eidx, ngh_times, time_w, time_b,
           Wq, Wk, Wv, W1, b1, W2, b2):
    fused = _fused_table(node_features, memory)

    nodes = jnp.concatenate([src_idx, tgt_idx, bgd_idx], axis=0)
    nodes3 = nodes.reshape(NW, RPW, LANE)
    nghT = ngh_nodes.T
    ngh3 = nghT.reshape(K, NW, RPW, LANE)
    eidx3 = ngh_eidx.T.reshape(K, NW, RPW, LANE)

    ngh_feat, ef, src_feat = _sc_gather(fused, edge_features, ngh3, eidx3, nodes3)

    ts = jnp.concatenate([cut_time, cut_time, cut_time], axis=0)
    dtT = ts[None, :] - ngh_times.T

    return _dense_stage(ngh_feat, ef, src_feat, dtT, nghT,
                        time_w, time_b, Wq, Wk, Wv, W1, b1, W2, b2)


# speedup vs baseline: 6.0546x; 1.0548x over previous
import functools

import jax
import jax.numpy as jnp
from jax import lax
from jax.experimental import pallas as pl
from jax.experimental.pallas import tpu as pltpu
from jax.experimental.pallas import tpu_sc as plsc

N = 100000
E = 1600000
D = 128
EDIM = 16
K = 20
B = 4096
H = 2
DH = D // H
B3 = 3 * B
NC = 2
NS = 16
NW = NC * NS
LANE = 128
RDIM = B3 // LANE
RPW = RDIM // NW

QBLK = 128


def _add_body(a_ref, b_ref, o_ref):
    o_ref[...] = a_ref[...] + b_ref[...]


def _fused_table(node_features, memory):
    rows = node_features.shape[0]
    blk = 2048
    grid = (rows + blk - 1) // blk
    return pl.pallas_call(
        _add_body,
        grid=(grid,),
        in_specs=[
            pl.BlockSpec((blk, D), lambda i: (i, 0)),
            pl.BlockSpec((blk, D), lambda i: (i, 0)),
        ],
        out_specs=pl.BlockSpec((blk, D), lambda i: (i, 0)),
        out_shape=jax.ShapeDtypeStruct((rows, D), jnp.float32),
    )(node_features, memory)


IRPW = (B3 * K) // (NW * LANE)
RING = 4
QRPW = B3 // (NW * LANE)


def _sc_gather_body(fused_hbm, etab_hbm, nghf_hbm, eidxf_hbm, nodes3_hbm,
                    ngh_out, ef_out, src_out,
                    nidx_v, eidx_v, qidx_v, rows_v, efrows_v,
                    gsem0, gsem1, gsem2, gsem3, esem0, esem1, esem2, esem3,
                    qsem):
    wid = lax.axis_index("s") * NC + lax.axis_index("c")
    cbase = wid * IRPW

    pltpu.sync_copy(nghf_hbm.at[wid], nidx_v)
    pltpu.sync_copy(eidxf_hbm.at[wid], eidx_v)
    pltpu.sync_copy(nodes3_hbm.at[wid], qidx_v)

    gsems = (gsem0, gsem1, gsem2, gsem3)
    esems = (esem0, esem1, esem2, esem3)

    def fire(i, b):
        pltpu.async_copy(fused_hbm.at[nidx_v.at[i]], rows_v.at[b], gsems[b])
        pltpu.async_copy(etab_hbm.at[eidx_v.at[i]], efrows_v.at[b], esems[b])

    def drain_write(i, b):
        pltpu.make_async_copy(ngh_out.at[0], rows_v.at[b], gsems[b]).wait()
        pltpu.sync_copy(rows_v.at[b], ngh_out.at[cbase + i])
        pltpu.make_async_copy(ef_out.at[0], efrows_v.at[b], esems[b]).wait()
        pltpu.sync_copy(efrows_v.at[b], ef_out.at[cbase + i])

    for c in range(QRPW):
        pltpu.async_copy(fused_hbm.at[qidx_v.at[c]],
                         rows_v.at[0], qsem).wait()
        pltpu.sync_copy(rows_v.at[0], src_out.at[wid * QRPW + c])

    for b in range(RING):
        fire(b, b)

    def body(h, _):
        i0 = RING * h
        for b in range(RING):
            drain_write(i0 + b, b)
            fire(i0 + RING + b, b)
        return 0

    lax.fori_loop(0, IRPW // RING - 1, body, 0)
    for b in range(RING):
        drain_write(IRPW - RING + b, b)


def _sc_gather(fused, edge_features, nghf, eidxf, nodes3):
    mesh = plsc.VectorSubcoreMesh(core_axis_name="c", subcore_axis_name="s")
    f = pl.kernel(
        _sc_gather_body,
        out_type=[
            jax.ShapeDtypeStruct((B3 * K // LANE, LANE, D), jnp.float32),
            jax.ShapeDtypeStruct((B3 * K // LANE, LANE, EDIM), jnp.float32),
            jax.ShapeDtypeStruct((B3 // LANE, LANE, D), jnp.float32),
        ],
        mesh=mesh,
        scratch_types=[
            pltpu.VMEM((IRPW, LANE), jnp.int32),
            pltpu.VMEM((IRPW, LANE), jnp.int32),
            pltpu.VMEM((QRPW, LANE), jnp.int32),
            pltpu.VMEM((RING, LANE, D), jnp.float32),
            pltpu.VMEM((RING, LANE, EDIM), jnp.float32),
            pltpu.SemaphoreType.DMA,
            pltpu.SemaphoreType.DMA,
            pltpu.SemaphoreType.DMA,
            pltpu.SemaphoreType.DMA,
            pltpu.SemaphoreType.DMA,
            pltpu.SemaphoreType.DMA,
            pltpu.SemaphoreType.DMA,
            pltpu.SemaphoreType.DMA,
            pltpu.SemaphoreType.DMA,
        ],
        compiler_params=pltpu.CompilerParams(use_tc_tiling_on_sc=False),
    )
    return f(fused, edge_features, nghf, eidxf, nodes3)


_INV2PI = 0.15915494309189535
_COS_C = (0.99999998, -19.73920414, 64.93912897, -85.45061558,
          60.17266892, -25.9887977, 6.56003795)


def _fast_cos(x):
    r = x * _INV2PI
    r = r - jnp.round(r)
    u = r * r
    p = jnp.float32(_COS_C[6])
    for c in (_COS_C[5], _COS_C[4], _COS_C[3], _COS_C[2], _COS_C[1], _COS_C[0]):
        p = p * u + jnp.float32(c)
    return p


def _dense_body(ngh_ref, ef_ref, src_ref, t_ref, nodes_ref, ts_ref,
                tw_ref, tb_ref, wq_ref, wk_ref, wv_ref, w1_ref, b1_ref,
                w2_ref, b2_ref, out_ref):
    q_n = QBLK
    tw = tw_ref[...]
    tb = tb_ref[...]

    dt = ts_ref[...] - t_ref[...]
    te = _fast_cos(dt[:, :, None] * tw[0][None, None, :] + tb[0][None, None, :])

    ngh2 = ngh_ref[...]
    ef2 = ef_ref[...]
    te2 = te.reshape(q_n * K, D)

    wk0 = wk_ref[0:D, :]
    wk1 = wk_ref[D:D + EDIM, :]
    wk2 = wk_ref[D + EDIM:D + EDIM + D, :]
    wv0 = wv_ref[0:D, :]
    wv1 = wv_ref[D:D + EDIM, :]
    wv2 = wv_ref[D + EDIM:D + EDIM + D, :]

    dot = functools.partial(jnp.dot, preferred_element_type=jnp.float32)
    kmat = dot(ngh2, wk0) + dot(ef2, wk1) + dot(te2, wk2)
    vmat = dot(ngh2, wv0) + dot(ef2, wv1) + dot(te2, wv2)

    src = src_ref[...]
    qbias = dot(jnp.cos(tb), wq_ref[D:2 * D, :])
    q = dot(src, wq_ref[0:D, :]) + qbias

    k3 = kmat.reshape(q_n, K, D)
    v3 = vmat.reshape(q_n, K, D)

    lane = lax.broadcasted_iota(jnp.int32, (1, 1, D), 2)
    m0 = (lane < DH).astype(jnp.float32)

    prod = q[:, None, :] * k3
    scale = 1.0 / jnp.sqrt(jnp.float32(DH))
    l_all = jnp.sum(prod, axis=-1) * scale
    l0 = jnp.sum(prod * m0, axis=-1) * scale
    l1 = l_all - l0

    invalid = nodes_ref[...] == 0
    neg = jnp.float32(-1e10)
    l0 = jnp.where(invalid, neg, l0)
    l1 = jnp.where(invalid, neg, l1)

    def softmax1(l):
        m = jnp.max(l, axis=1, keepdims=True)
        e = jnp.exp(l - m)
        return e / jnp.sum(e, axis=1, keepdims=True)

    a0 = softmax1(l0)
    a1 = softmax1(l1)
    af = a0[:, :, None] * m0 + a1[:, :, None] * (1.0 - m0)
    attn_out = jnp.sum(af * v3, axis=1)

    hmid = jnp.maximum(
        dot(attn_out, w1_ref[0:D, :]) + dot(src, w1_ref[D:2 * D, :]) + b1_ref[...],
        0.0)
    out_ref[...] = dot(hmid, w2_ref[...]) + b2_ref[...]


def _dense_stage(ngh_flat, ef_flat, src_feat, ngh_times, ngh_nodes, ts2,
                 time_w, time_b, Wq, Wk, Wv, W1, b1, W2, b2):
    grid = (B3 // QBLK,)

    def wspec(shape):
        nd = len(shape)
        return pl.BlockSpec(shape, lambda i, _nd=nd: (0,) * _nd)

    return pl.pallas_call(
        _dense_body,
        grid=grid,
        in_specs=[
            pl.BlockSpec((QBLK * K, D), lambda i: (i, 0)),
            pl.BlockSpec((QBLK * K, EDIM), lambda i: (i, 0)),
            pl.BlockSpec((QBLK, D), lambda i: (i, 0)),
            pl.BlockSpec((QBLK, K), lambda i: (i, 0)),
            pl.BlockSpec((QBLK, K), lambda i: (i, 0)),
            pl.BlockSpec((QBLK, 1), lambda i: (i, 0)),
            wspec((1, D)),
            wspec((1, D)),
            wspec((2 * D, D)),
            wspec((2 * D + EDIM, D)),
            wspec((2 * D + EDIM, D)),
            wspec((2 * D, D)),
            wspec((1, D)),
            wspec((D, D)),
            wspec((1, D)),
        ],
        out_specs=pl.BlockSpec((QBLK, D), lambda i: (i, 0)),
        out_shape=jax.ShapeDtypeStruct((B3, D), jnp.float32),
    )(ngh_flat, ef_flat, src_feat, ngh_times, ngh_nodes, ts2,
      time_w.reshape(1, D), time_b.reshape(1, D), Wq, Wk, Wv,
      W1, b1.reshape(1, D), W2, b2.reshape(1, D))


def kernel(node_features, edge_features, memory, src_idx, tgt_idx, bgd_idx,
           cut_time, e_idx, ngh_nodes, ngh_eidx, ngh_times, time_w, time_b,
           Wq, Wk, Wv, W1, b1, W2, b2):
    fused = _fused_table(node_features, memory)

    nodes = jnp.concatenate([src_idx, tgt_idx, bgd_idx], axis=0)
    nodes3 = nodes.reshape(NW, QRPW, LANE)
    nghf = ngh_nodes.reshape(NW, IRPW, LANE)
    eidxf = ngh_eidx.reshape(NW, IRPW, LANE)

    ngh3d, ef3d, src3d = _sc_gather(fused, edge_features, nghf, eidxf, nodes3)
    ngh_flat = ngh3d.reshape(B3 * K, D)
    ef_flat = ef3d.reshape(B3 * K, EDIM)
    src_feat = src3d.reshape(B3, D)

    ts2 = jnp.concatenate([cut_time, cut_time, cut_time], axis=0).reshape(B3, 1)

    return _dense_stage(ngh_flat, ef_flat, src_feat, ngh_times, ngh_nodes, ts2,
                        time_w, time_b, Wq, Wk, Wv, W1, b1, W2, b2)
